# baseline (speedup 1.0000x reference)
import functools

import jax
import jax.numpy as jnp
from jax import lax
from jax.experimental import pallas as pl
from jax.experimental.pallas import tpu as pltpu
from jax.experimental.pallas import tpu_sc as plsc

_FH, _FW = 3, 5
_S = 14
_WIN = _FH * _FW
_W = 224
_SPAN = 672


def _body(xx_hbm, out_hbm, buf, stage):
    sid = lax.axis_index("s")
    lane = lax.iota(jnp.int32, 16)
    cols = _S * lane
    for r in range(2):
        i = sid * 2 + r
        base = (_S * i) * _W
        pltpu.sync_copy(xx_hbm.at[pl.ds(base, _SPAN)], buf)
        acc = jnp.zeros((16,), jnp.float32)
        for dy in range(_FH):
            for dx in range(_FW):
                vals = plsc.load_gather(buf, [cols + (dy * _W + dx)])
                acc = acc + jnp.where(vals <= 0.0, 1.0, 0.0)
        plsc.store_scatter(stage, [2 * lane], acc)
        plsc.store_scatter(stage, [2 * lane + 1], float(_WIN) - acc)
        pltpu.sync_copy(stage, out_hbm.at[pl.ds(i * 32, 32)])


@functools.cache
def _k():
    return functools.partial(
        pl.kernel,
        out_type=jax.ShapeDtypeStruct((512,), jnp.float32),
        mesh=plsc.VectorSubcoreMesh(
            core_axis_name="c", subcore_axis_name="s", num_cores=1, num_subcores=8
        ),
        compiler_params=pltpu.CompilerParams(needs_layout_passes=False, disable_bounds_checks=True, disable_semaphore_checks=True),
        scratch_types=[
            pltpu.VMEM((_SPAN,), jnp.float32),
            pltpu.VMEM((32,), jnp.float32),
        ],
    )(_body)


def kernel(xx):
    return _k()(xx.reshape(-1)).reshape(16, 16, 2)


# speedup vs baseline: 1.0338x; 1.0338x over previous
import functools

import jax
import jax.numpy as jnp
from jax import lax
from jax.experimental import pallas as pl
from jax.experimental.pallas import tpu as pltpu
from jax.experimental.pallas import tpu_sc as plsc

_FH, _FW = 3, 5
_S = 14
_WIN = _FH * _FW
_W = 224
_SPAN = 672


def _body(xx_hbm, out_hbm, buf, stage):
    i = lax.axis_index("s")
    base = (_S * i) * _W
    pltpu.sync_copy(xx_hbm.at[pl.ds(base, _SPAN)], buf)

    lane = lax.iota(jnp.int32, 16)
    cols = _S * lane
    acc = jnp.zeros((16,), jnp.float32)
    for dy in range(_FH):
        for dx in range(_FW):
            vals = plsc.load_gather(buf, [cols + (dy * _W + dx)])
            acc = acc + jnp.where(vals <= 0.0, 1.0, 0.0)

    plsc.store_scatter(stage, [2 * lane], acc)
    plsc.store_scatter(stage, [2 * lane + 1], float(_WIN) - acc)
    pltpu.sync_copy(stage, out_hbm.at[pl.ds(i * 32, 32)])


@functools.cache
def _k():
    return functools.partial(
        pl.kernel,
        out_type=jax.ShapeDtypeStruct((512,), jnp.float32),
        mesh=plsc.VectorSubcoreMesh(
            core_axis_name="c", subcore_axis_name="s", num_cores=1
        ),
        compiler_params=pltpu.CompilerParams(needs_layout_passes=False, disable_bounds_checks=True, disable_semaphore_checks=True),
        scratch_types=[
            pltpu.VMEM((_SPAN,), jnp.float32),
            pltpu.VMEM((32,), jnp.float32),
        ],
    )(_body)


def kernel(xx):
    return _k()(xx.reshape(-1)).reshape(16, 16, 2)
